# baseline (speedup 1.0000x reference)
import functools

import jax
import jax.numpy as jnp
from jax import lax
from jax.experimental import pallas as pl
from jax.experimental.pallas import tpu as pltpu

_TILE_B = 2048


def _critic_kernel(state_ref, action_ref, w0s_ref, w0a_ref, w1a_ref, w1b_ref,
                   wl1_ref, wl2_ref, b_ref, b_last_ref, out1_ref, out2_ref,
                   *, H):
    b = b_ref[...]

    s = state_ref[...].astype(jnp.bfloat16)
    a = action_ref[...].astype(jnp.bfloat16)
    h = (jnp.dot(s, w0s_ref[...], preferred_element_type=jnp.float32)
         + jnp.dot(a, w0a_ref[...], preferred_element_type=jnp.float32)
         + b[0:1, :])
    h = jnp.maximum(h, 0.0).astype(jnp.bfloat16)

    g1 = jnp.maximum(
        jnp.dot(h[:, :H], w1a_ref[...], preferred_element_type=jnp.float32)
        + b[1:2, :H], 0.0).astype(jnp.bfloat16)
    g2 = jnp.maximum(
        jnp.dot(h[:, H:], w1b_ref[...], preferred_element_type=jnp.float32)
        + b[1:2, H:], 0.0).astype(jnp.bfloat16)

    q1 = lax.dot_general(wl1_ref[...], g1,
                         dimension_numbers=(((0,), (1,)), ((), ())),
                         preferred_element_type=jnp.float32)
    q2 = lax.dot_general(wl2_ref[...], g2,
                         dimension_numbers=(((0,), (1,)), ((), ())),
                         preferred_element_type=jnp.float32)
    b_last = b_last_ref[...]
    out1_ref[...] = q1.T + b_last[0, 0]
    out2_ref[...] = q2.T + b_last[1, 0]


@jax.jit
def _forward(state, action, w0_s, w0_a, hidden_w0, w_last, biases, b_last):
    B, S = state.shape
    A = action.shape[1]
    H = hidden_w0.shape[0] // 2

    w0s = w0_s.astype(jnp.bfloat16)
    w0a = w0_a.astype(jnp.bfloat16)
    w1a = hidden_w0[:H, :H].astype(jnp.bfloat16)
    w1b = hidden_w0[H:, H:].astype(jnp.bfloat16)
    wl1 = w_last[:H, 0:1].astype(jnp.bfloat16)
    wl2 = w_last[H:, 1:2].astype(jnp.bfloat16)

    tile = _TILE_B
    num_tiles = pl.cdiv(B, tile)
    b_pad = num_tiles * tile
    if b_pad != B:
        state = jnp.pad(state, ((0, b_pad - B), (0, 0)))
        action = jnp.pad(action, ((0, b_pad - B), (0, 0)))

    weight_inputs = [w0s, w0a, w1a, w1b, wl1, wl2, biases, b_last]

    def resident(arr):
        nd = arr.ndim
        return pl.BlockSpec(arr.shape, lambda i: (0,) * nd)

    q1, q2 = pl.pallas_call(
        functools.partial(_critic_kernel, H=H),
        grid=(num_tiles,),
        out_shape=[jax.ShapeDtypeStruct((b_pad, 1), jnp.float32),
                   jax.ShapeDtypeStruct((b_pad, 1), jnp.float32)],
        in_specs=[
            pl.BlockSpec((tile, S), lambda i: (i, 0)),
            pl.BlockSpec((tile, A), lambda i: (i, 0)),
            *[resident(w) for w in weight_inputs],
        ],
        out_specs=[pl.BlockSpec((tile, 1), lambda i: (i, 0)),
                   pl.BlockSpec((tile, 1), lambda i: (i, 0))],
        compiler_params=pltpu.CompilerParams(
            dimension_semantics=("parallel",)),
    )(state, action, *weight_inputs)

    if b_pad != B:
        q1, q2 = q1[:B], q2[:B]
    return q1, q2


def kernel(state, action, w0_s, w0_a, hidden_w0, w_last, biases, b_last):
    return _forward(state, action, w0_s, w0_a, hidden_w0, w_last, biases,
                    b_last)


# speedup vs baseline: 1.0580x; 1.0580x over previous
import functools

import jax
import jax.numpy as jnp
from jax import lax
from jax.experimental import pallas as pl
from jax.experimental.pallas import tpu as pltpu

_TILE_B = 4096


_N_CHUNKS = 16


def _critic_kernel(state_ref, action_ref, w0s_ref, w0a_ref, hidden_ref,
                   wlast_ref, b_ref, b_last_ref, out1_ref, out2_ref, *, H):
    b = b_ref[...]
    b_last = b_last_ref[...]

    C = out1_ref.shape[0] // _N_CHUNKS
    for c in range(_N_CHUNKS):
        rows = pl.ds(c * C, C)

        h = (jnp.dot(state_ref[rows, :], w0s_ref[...],
                     preferred_element_type=jnp.float32)
             + jnp.dot(action_ref[rows, :], w0a_ref[...],
                       preferred_element_type=jnp.float32)
             + b[0:1, :])
        h = jnp.maximum(h, 0.0)

        g1 = jnp.maximum(
            jnp.dot(h[:, :H], hidden_ref[:H, :H],
                    preferred_element_type=jnp.float32) + b[1:2, :H], 0.0)
        g2 = jnp.maximum(
            jnp.dot(h[:, H:], hidden_ref[H:, H:],
                    preferred_element_type=jnp.float32) + b[1:2, H:], 0.0)

        q1 = lax.dot_general(wlast_ref[:H, 0:1], g1,
                             dimension_numbers=(((0,), (1,)), ((), ())),
                             preferred_element_type=jnp.float32)
        q2 = lax.dot_general(wlast_ref[H:, 1:2], g2,
                             dimension_numbers=(((0,), (1,)), ((), ())),
                             preferred_element_type=jnp.float32)
        out1_ref[rows, :] = q1.T + b_last[0, 0]
        out2_ref[rows, :] = q2.T + b_last[1, 0]


@jax.jit
def _forward(state, action, w0_s, w0_a, hidden_w0, w_last, biases, b_last):
    B, S = state.shape
    A = action.shape[1]
    H = hidden_w0.shape[0] // 2

    tile = _TILE_B
    num_tiles = pl.cdiv(B, tile)
    b_pad = num_tiles * tile
    if b_pad != B:
        state = jnp.pad(state, ((0, b_pad - B), (0, 0)))
        action = jnp.pad(action, ((0, b_pad - B), (0, 0)))

    weight_inputs = [w0_s, w0_a, hidden_w0, w_last, biases, b_last]

    def resident(arr):
        nd = arr.ndim
        return pl.BlockSpec(arr.shape, lambda i: (0,) * nd)

    q1, q2 = pl.pallas_call(
        functools.partial(_critic_kernel, H=H),
        grid=(num_tiles,),
        out_shape=[jax.ShapeDtypeStruct((b_pad, 1), jnp.float32),
                   jax.ShapeDtypeStruct((b_pad, 1), jnp.float32)],
        in_specs=[
            pl.BlockSpec((tile, S), lambda i: (i, 0)),
            pl.BlockSpec((tile, A), lambda i: (i, 0)),
            *[resident(w) for w in weight_inputs],
        ],
        out_specs=[pl.BlockSpec((tile, 1), lambda i: (i, 0)),
                   pl.BlockSpec((tile, 1), lambda i: (i, 0))],
        compiler_params=pltpu.CompilerParams(
            dimension_semantics=("parallel",)),
    )(state, action, *weight_inputs)

    if b_pad != B:
        q1, q2 = q1[:B], q2[:B]
    return q1, q2


def kernel(state, action, w0_s, w0_a, hidden_w0, w_last, biases, b_last):
    return _forward(state, action, w0_s, w0_a, hidden_w0, w_last, biases,
                    b_last)


# speedup vs baseline: 1.8473x; 1.7460x over previous
import functools

import jax
import jax.numpy as jnp
from jax import lax
from jax.experimental import pallas as pl
from jax.experimental.pallas import tpu as pltpu

_TILE_B = 4096
_N_CHUNKS = 4


def _critic_kernel(state_ref, action_t_ref, w0s_ref, w0a_ref, hidden_ref,
                   wlast_ref, b_ref, b_last_ref, out_ref, *, H):
    b = b_ref[...]
    b_last = b_last_ref[...]

    C = out_ref.shape[1] // _N_CHUNKS
    for c in range(_N_CHUNKS):
        rows = pl.ds(c * C, C)

        h = (jnp.dot(state_ref[rows, :], w0s_ref[...],
                     preferred_element_type=jnp.float32)
             + lax.dot_general(action_t_ref[:, rows], w0a_ref[...],
                               dimension_numbers=(((0,), (0,)), ((), ())),
                               preferred_element_type=jnp.float32)
             + b[0:1, :])
        h = jnp.maximum(h, 0.0)

        g1 = jnp.maximum(
            jnp.dot(h[:, :H], hidden_ref[:H, :H],
                    preferred_element_type=jnp.float32) + b[1:2, :H], 0.0)
        g2 = jnp.maximum(
            jnp.dot(h[:, H:], hidden_ref[H:, H:],
                    preferred_element_type=jnp.float32) + b[1:2, H:], 0.0)

        q1 = lax.dot_general(wlast_ref[:H, 0:1], g1,
                             dimension_numbers=(((0,), (1,)), ((), ())),
                             preferred_element_type=jnp.float32)
        q2 = lax.dot_general(wlast_ref[H:, 1:2], g2,
                             dimension_numbers=(((0,), (1,)), ((), ())),
                             preferred_element_type=jnp.float32)
        out_ref[:, rows] = jnp.concatenate([q1, q2], axis=0) + b_last


@jax.jit
def _forward(state, action, w0_s, w0_a, hidden_w0, w_last, biases, b_last):
    B, S = state.shape
    A = action.shape[1]
    H = hidden_w0.shape[0] // 2

    tile = _TILE_B
    num_tiles = pl.cdiv(B, tile)
    b_pad = num_tiles * tile
    if b_pad != B:
        state = jnp.pad(state, ((0, b_pad - B), (0, 0)))
        action = jnp.pad(action, ((0, b_pad - B), (0, 0)))

    action_t = action.T

    weight_inputs = [w0_s, w0_a, hidden_w0, w_last, biases, b_last]

    def resident(arr):
        nd = arr.ndim
        return pl.BlockSpec(arr.shape, lambda i: (0,) * nd)

    qs = pl.pallas_call(
        functools.partial(_critic_kernel, H=H),
        grid=(num_tiles,),
        out_shape=jax.ShapeDtypeStruct((2, b_pad), jnp.float32),
        in_specs=[
            pl.BlockSpec((tile, S), lambda i: (i, 0)),
            pl.BlockSpec((A, tile), lambda i: (0, i)),
            *[resident(w) for w in weight_inputs],
        ],
        out_specs=pl.BlockSpec((2, tile), lambda i: (0, i)),
        compiler_params=pltpu.CompilerParams(
            dimension_semantics=("parallel",)),
    )(state, action_t, *weight_inputs)

    return qs[0, :B].reshape(B, 1), qs[1, :B].reshape(B, 1)


def kernel(state, action, w0_s, w0_a, hidden_w0, w_last, biases, b_last):
    return _forward(state, action, w0_s, w0_a, hidden_w0, w_last, biases,
                    b_last)
